# baseline (speedup 1.0000x reference)
import functools

import jax
import jax.numpy as jnp
import numpy as np
from jax import lax
from jax.experimental import pallas as pl
from jax.experimental.pallas import tpu as pltpu
from jax.experimental.pallas import tpu_sc as plsc

N = 10000
E = 320000
D = 128
H = D // 2

NC = 2
NS = 16
NW = NC * NS
C = 80

DPW = E // NW
DCH = DPW // C

ECH = 260
EPAD = NS * ECH * C

SL = 624
TAIL = N - NS * SL

_MESH = plsc.VectorSubcoreMesh(
    core_axis_name="c", subcore_axis_name="s", num_cores=NC, num_subcores=NS
)


@functools.partial(
    pl.kernel,
    out_type=jax.ShapeDtypeStruct((NC, N), jnp.float32),
    mesh=_MESH,
    scratch_types=[
        pltpu.VMEM((DCH, C), jnp.int32),
        pltpu.VMEM((DCH, C), jnp.float32),
        pltpu.VMEM_SHARED((N,), jnp.float32),
    ],
)
def _deg(dst_hbm, ew_hbm, zn_hbm, out_hbm, dstv, ewv, degsh):
    ci = lax.axis_index("c")
    si = lax.axis_index("s")
    wid = ci * NS + si
    pltpu.sync_copy(dst_hbm.at[wid], dstv)
    pltpu.sync_copy(ew_hbm.at[wid], ewv)

    @pl.when(si == 0)
    def _():
        pltpu.sync_copy(zn_hbm, degsh)

    plsc.subcore_barrier()

    def body(j, carry):
        pltpu.sync_copy(ewv.at[j], degsh.at[dstv.at[j]], add=True)
        return carry

    lax.fori_loop(0, DCH, body, 0)
    plsc.subcore_barrier()

    @pl.when(si == 0)
    def _():
        pltpu.sync_copy(degsh, out_hbm.at[ci])


@functools.partial(
    pl.kernel,
    out_type=jax.ShapeDtypeStruct((NC, N, H), jnp.float32),
    mesh=_MESH,
    scratch_types=[
        pltpu.VMEM((ECH, C), jnp.int32),
        pltpu.VMEM((ECH, C), jnp.int32),
        pltpu.VMEM((5, C), jnp.float32),
        pltpu.VMEM((5, C, H // 2), jnp.int32),
        pltpu.VMEM((5, C, H), jnp.float32),
        pltpu.VMEM_SHARED((N, H), jnp.float32),
        pltpu.SemaphoreType.DMA,
        pltpu.SemaphoreType.DMA,
    ],
    compiler_params=pltpu.CompilerParams(use_tc_tiling_on_sc=False),
)
def _edge(h_hbm, src_hbm, dst_hbm, ew_hbm, znd_hbm, out_hbm,
          srcv, dstv, eww, rows, sbuf, acc, gsem, ssem):
    ci = lax.axis_index("c")
    si = lax.axis_index("s")
    pltpu.sync_copy(src_hbm.at[si], srcv)
    pltpu.sync_copy(dst_hbm.at[si], dstv)
    pltpu.sync_copy(znd_hbm.at[pl.ds(si * SL, SL)], acc.at[pl.ds(si * SL, SL)])

    @pl.when(si == 0)
    def _():
        pltpu.sync_copy(znd_hbm.at[pl.ds(NS * SL, TAIL)],
                        acc.at[pl.ds(NS * SL, TAIL)])

    off = (ci * N).astype(jnp.int32)

    def offset_body(j, carry):
        for g in range(C // 16):
            srcv[j, 16 * g:16 * (g + 1)] = srcv[j, 16 * g:16 * (g + 1)] + off
        return carry

    lax.fori_loop(0, ECH, offset_body, 0)
    plsc.subcore_barrier()

    NB = 5
    PF = 4
    for p in range(PF):
        pltpu.async_copy(h_hbm.at[srcv.at[p]], rows.at[p], gsem)
        pltpu.async_copy(ew_hbm.at[si, p], eww.at[p], gsem)

    def chunk(j, b):
        pltpu.make_async_copy(h_hbm.at[srcv.at[j]], rows.at[b], gsem).wait()
        pltpu.make_async_copy(ew_hbm.at[si, j], eww.at[b], gsem).wait()

        @pl.when(j >= NB - PF)
        def _():
            pltpu.make_async_copy(sbuf.at[(b + PF) % NB],
                                  acc.at[dstv.at[j - (NB - PF)]], ssem).wait()

        @pl.when(j + PF < ECH)
        def _():
            pltpu.async_copy(h_hbm.at[srcv.at[j + PF]],
                             rows.at[(b + PF) % NB], gsem)
            pltpu.async_copy(ew_hbm.at[si, j + PF],
                             eww.at[(b + PF) % NB], gsem)

        for g in range(C // 16):
            ewg = eww[b, 16 * g:16 * (g + 1)]
            for l in range(16):
                wv = lax.gather(
                    ewg, jnp.full((16, 1), l, jnp.int32),
                    lax.GatherDimensionNumbers(offset_dims=(),
                                               collapsed_slice_dims=(0,),
                                               start_index_map=(0,)),
                    slice_sizes=(1,),
                    mode=lax.GatherScatterMode.PROMISE_IN_BOUNDS)
                e = 16 * g + l
                for k in range(H // 32):
                    vi = rows[b, e, 16 * k:16 * (k + 1)]
                    lo = lax.bitcast_convert_type(vi << 16, jnp.float32)
                    hi = lax.bitcast_convert_type(vi & jnp.int32(-65536),
                                                  jnp.float32)
                    sbuf[b, e, 16 * k:16 * (k + 1)] = lo * wv
                    sbuf[b, e, 32 + 16 * k:32 + 16 * (k + 1)] = hi * wv
        pltpu.async_copy(sbuf.at[b], acc.at[dstv.at[j]], ssem, add=True)

    def outer(jo, carry):
        for b in range(NB):
            chunk(NB * jo + b, b)
        return carry

    lax.fori_loop(0, ECH // NB, outer, 0)
    for t in range(NB - PF):
        pltpu.make_async_copy(sbuf.at[(ECH - 1 - t) % NB],
                              acc.at[dstv.at[ECH - 1 - t]], ssem).wait()
    plsc.subcore_barrier()
    pltpu.sync_copy(acc.at[pl.ds(si * SL, SL)],
                    out_hbm.at[ci, pl.ds(si * SL, SL)])

    @pl.when(si == 0)
    def _():
        pltpu.sync_copy(acc.at[pl.ds(NS * SL, TAIL)],
                        out_hbm.at[ci, pl.ds(NS * SL, TAIL)])


_RB = 2000
_GRID = N // _RB


def _dis_of(degT_blk):
    deg = degT_blk[:, 0:1] + degT_blk[:, 1:2] + 1.0
    return lax.rsqrt(deg)


def _split_store(out_ref, val):
    u = lax.bitcast_convert_type(val.astype(jnp.bfloat16), jnp.uint16)
    u = u.astype(jnp.uint32)
    for c in range(NC):
        lo = u[:, c * H:c * H + 32]
        hi = u[:, c * H + 32:(c + 1) * H]
        out_ref[c] = lax.bitcast_convert_type(lo | (hi << 16), jnp.int32)


def _cat(p_ref):
    return jnp.concatenate([p_ref[0], p_ref[1]], axis=1)


def _tc1_body(x_ref, w1_ref, degT_ref, out_ref):
    dis = _dis_of(degT_ref[...])
    _split_store(out_ref, jnp.dot(x_ref[...], w1_ref[...],
                                  preferred_element_type=jnp.float32) * dis)


def _tc2_body(p_ref, degT_ref, b1_ref, g_ref, be_ref, mu_ref, va_ref,
              w2_ref, out_ref):
    dis = _dis_of(degT_ref[...])
    t = dis * _cat(p_ref) + b1_ref[...]
    inv = lax.rsqrt(va_ref[...] + 1e-5)
    t = (t - mu_ref[...]) * inv * g_ref[...] + be_ref[...]
    t = jnp.maximum(t, 0.0)
    _split_store(out_ref, jnp.dot(t, w2_ref[...],
                                  preferred_element_type=jnp.float32) * dis)


def _tc3_body(p_ref, degT_ref, b2_ref, out_ref):
    dis = _dis_of(degT_ref[...])
    out_ref[...] = dis * _cat(p_ref) + b2_ref[...]


_rowspec = pl.BlockSpec((_RB, D), lambda i: (i, 0))
_fullmat = pl.BlockSpec((D, D), lambda i: (0, 0))
_degspec = pl.BlockSpec((_RB, NC), lambda i: (i, 0))
_vecspec = pl.BlockSpec((1, D), lambda i: (0, 0))
_halfspec = pl.BlockSpec((NC, _RB, H), lambda i: (0, i, 0))
_packspec = pl.BlockSpec((NC, _RB, H // 2), lambda i: (0, i, 0))

_tc1 = pl.pallas_call(
    _tc1_body,
    grid=(_GRID,),
    in_specs=[_rowspec, _fullmat, _degspec],
    out_specs=_packspec,
    out_shape=jax.ShapeDtypeStruct((NC, N, H // 2), jnp.int32),
)

_tc2 = pl.pallas_call(
    _tc2_body,
    grid=(_GRID,),
    in_specs=[_halfspec, _degspec,
              _vecspec, _vecspec, _vecspec, _vecspec, _vecspec, _fullmat],
    out_specs=_packspec,
    out_shape=jax.ShapeDtypeStruct((NC, N, H // 2), jnp.int32),
)

_tc3 = pl.pallas_call(
    _tc3_body,
    grid=(_GRID,),
    in_specs=[_halfspec, _degspec, _vecspec],
    out_specs=_rowspec,
    out_shape=jax.ShapeDtypeStruct((N, D), jnp.float32),
)


def kernel(x, edge_index, edge_weight, W1, b1, bn_gamma, bn_beta, bn_mean,
           bn_var, W2, b2):
    loop = jnp.arange(N, dtype=jnp.int32)
    padn = EPAD - E - N
    pad = (jnp.arange(padn, dtype=jnp.int32) * 37) % N
    srcf = jnp.concatenate([edge_index[0], loop, pad]).reshape(NS, ECH, C)
    dstf = jnp.concatenate([edge_index[1], loop, pad]).reshape(NS, ECH, C)
    ewf = jnp.concatenate([
        edge_weight, jnp.ones((N,), jnp.float32),
        jnp.zeros((padn,), jnp.float32)]).reshape(NS, ECH, C)

    dstd = edge_index[1].reshape(NW, DCH, C)
    ewd = edge_weight.reshape(NW, DCH, C)
    zn = jnp.zeros((N,), jnp.float32)
    znd = jnp.zeros((N, H), jnp.float32)

    b1s = b1.reshape(1, D)
    gs = bn_gamma.reshape(1, D)
    bes = bn_beta.reshape(1, D)
    mus = bn_mean.reshape(1, D)
    vas = bn_var.reshape(1, D)
    b2s = b2.reshape(1, D)

    degP = _deg(dstd, ewd, zn)
    degT = degP.T

    h1 = _tc1(x, W1, degT)
    p1 = _edge(h1.reshape(NC * N, H // 2), srcf, dstf, ewf, znd)
    h2 = _tc2(p1, degT, b1s, gs, bes, mus, vas, W2)
    p2 = _edge(h2.reshape(NC * N, H // 2), srcf, dstf, ewf, znd)
    return _tc3(p2, degT, b2s)


# speedup vs baseline: 1.0604x; 1.0604x over previous
import functools

import jax
import jax.numpy as jnp
import numpy as np
from jax import lax
from jax.experimental import pallas as pl
from jax.experimental.pallas import tpu as pltpu
from jax.experimental.pallas import tpu_sc as plsc

N = 10000
E = 320000
D = 128
H = D // 2

NC = 2
NS = 16
NW = NC * NS
C = 80

DPW = E // NW
DCH = DPW // C

ECH = 260
EPAD = NS * ECH * C

SL = 624
TAIL = N - NS * SL

_MESH = plsc.VectorSubcoreMesh(
    core_axis_name="c", subcore_axis_name="s", num_cores=NC, num_subcores=NS
)


@functools.partial(
    pl.kernel,
    out_type=jax.ShapeDtypeStruct((NC, N), jnp.float32),
    mesh=_MESH,
    scratch_types=[
        pltpu.VMEM((DCH, C), jnp.int32),
        pltpu.VMEM((DCH, C), jnp.float32),
        pltpu.VMEM_SHARED((N,), jnp.float32),
    ],
)
def _deg(dst_hbm, ew_hbm, zn_hbm, out_hbm, dstv, ewv, degsh):
    ci = lax.axis_index("c")
    si = lax.axis_index("s")
    wid = ci * NS + si
    pltpu.sync_copy(dst_hbm.at[wid], dstv)
    pltpu.sync_copy(ew_hbm.at[wid], ewv)

    @pl.when(si == 0)
    def _():
        pltpu.sync_copy(zn_hbm, degsh)

    plsc.subcore_barrier()

    def body(j, carry):
        pltpu.sync_copy(ewv.at[j], degsh.at[dstv.at[j]], add=True)
        return carry

    lax.fori_loop(0, DCH, body, 0)
    plsc.subcore_barrier()

    @pl.when(si == 0)
    def _():
        pltpu.sync_copy(degsh, out_hbm.at[ci])


@functools.partial(
    pl.kernel,
    out_type=jax.ShapeDtypeStruct((NC, N, H), jnp.float32),
    mesh=_MESH,
    scratch_types=[
        pltpu.VMEM((ECH, C), jnp.int32),
        pltpu.VMEM((ECH, C), jnp.int32),
        pltpu.VMEM((5, C), jnp.float32),
        pltpu.VMEM((5, C, H // 2), jnp.int32),
        pltpu.VMEM((5, C, H), jnp.float32),
        pltpu.VMEM_SHARED((N, H), jnp.float32),
        pltpu.SemaphoreType.DMA,
        pltpu.SemaphoreType.DMA,
    ],
    compiler_params=pltpu.CompilerParams(use_tc_tiling_on_sc=False),
)
def _edge(h_hbm, src_hbm, dst_hbm, ew_hbm, znd_hbm, out_hbm,
          srcv, dstv, eww, rows, sbuf, acc, gsem, ssem):
    ci = lax.axis_index("c")
    si = lax.axis_index("s")
    pltpu.sync_copy(src_hbm.at[si], srcv)
    pltpu.sync_copy(dst_hbm.at[si], dstv)
    pltpu.sync_copy(znd_hbm.at[pl.ds(si * SL, SL)], acc.at[pl.ds(si * SL, SL)])

    @pl.when(si == 0)
    def _():
        pltpu.sync_copy(znd_hbm.at[pl.ds(NS * SL, TAIL)],
                        acc.at[pl.ds(NS * SL, TAIL)])

    off = (ci * N).astype(jnp.int32)

    def offset_body(j, carry):
        for g in range(C // 16):
            srcv[j, 16 * g:16 * (g + 1)] = srcv[j, 16 * g:16 * (g + 1)] + off
        return carry

    lax.fori_loop(0, ECH, offset_body, 0)
    plsc.subcore_barrier()

    NB = 5
    PF = 3
    for p in range(PF):
        pltpu.async_copy(h_hbm.at[srcv.at[p]], rows.at[p], gsem)
        pltpu.async_copy(ew_hbm.at[si, p], eww.at[p], gsem)

    def chunk(j, b):
        pltpu.make_async_copy(h_hbm.at[srcv.at[j]], rows.at[b], gsem).wait()
        pltpu.make_async_copy(ew_hbm.at[si, j], eww.at[b], gsem).wait()

        @pl.when(j + PF < ECH)
        def _():
            pltpu.async_copy(h_hbm.at[srcv.at[j + PF]],
                             rows.at[(b + PF) % NB], gsem)
            pltpu.async_copy(ew_hbm.at[si, j + PF],
                             eww.at[(b + PF) % NB], gsem)

        @pl.when(j >= NB - PF)
        def _():
            pltpu.make_async_copy(sbuf.at[(b + PF) % NB],
                                  acc.at[dstv.at[j - (NB - PF)]], ssem).wait()

        for g in range(C // 16):
            ewg = eww[b, 16 * g:16 * (g + 1)]
            for l in range(16):
                wv = lax.gather(
                    ewg, jnp.full((16, 1), l, jnp.int32),
                    lax.GatherDimensionNumbers(offset_dims=(),
                                               collapsed_slice_dims=(0,),
                                               start_index_map=(0,)),
                    slice_sizes=(1,),
                    mode=lax.GatherScatterMode.PROMISE_IN_BOUNDS)
                e = 16 * g + l
                for k in range(H // 32):
                    vi = rows[b, e, 16 * k:16 * (k + 1)]
                    lo = lax.bitcast_convert_type(vi << 16, jnp.float32)
                    hi = lax.bitcast_convert_type(vi & jnp.int32(-65536),
                                                  jnp.float32)
                    sbuf[b, e, 16 * k:16 * (k + 1)] = lo * wv
                    sbuf[b, e, 32 + 16 * k:32 + 16 * (k + 1)] = hi * wv
        pltpu.async_copy(sbuf.at[b], acc.at[dstv.at[j]], ssem, add=True)

    def outer(jo, carry):
        for b in range(NB):
            chunk(NB * jo + b, b)
        return carry

    lax.fori_loop(0, ECH // NB, outer, 0)
    for t in range(NB - PF):
        pltpu.make_async_copy(sbuf.at[(ECH - 1 - t) % NB],
                              acc.at[dstv.at[ECH - 1 - t]], ssem).wait()
    plsc.subcore_barrier()
    pltpu.sync_copy(acc.at[pl.ds(si * SL, SL)],
                    out_hbm.at[ci, pl.ds(si * SL, SL)])

    @pl.when(si == 0)
    def _():
        pltpu.sync_copy(acc.at[pl.ds(NS * SL, TAIL)],
                        out_hbm.at[ci, pl.ds(NS * SL, TAIL)])


_RB = 2000
_GRID = N // _RB


def _dis_of(degT_blk):
    deg = degT_blk[:, 0:1] + degT_blk[:, 1:2] + 1.0
    return lax.rsqrt(deg)


def _split_store(out_ref, val):
    u = lax.bitcast_convert_type(val.astype(jnp.bfloat16), jnp.uint16)
    u = u.astype(jnp.uint32)
    for c in range(NC):
        lo = u[:, c * H:c * H + 32]
        hi = u[:, c * H + 32:(c + 1) * H]
        out_ref[c] = lax.bitcast_convert_type(lo | (hi << 16), jnp.int32)


def _cat(p_ref):
    return jnp.concatenate([p_ref[0], p_ref[1]], axis=1)


def _tc1_body(x_ref, w1_ref, degT_ref, out_ref):
    dis = _dis_of(degT_ref[...])
    _split_store(out_ref, jnp.dot(x_ref[...], w1_ref[...],
                                  preferred_element_type=jnp.float32) * dis)


def _tc2_body(p_ref, degT_ref, b1_ref, g_ref, be_ref, mu_ref, va_ref,
              w2_ref, out_ref):
    dis = _dis_of(degT_ref[...])
    t = dis * _cat(p_ref) + b1_ref[...]
    inv = lax.rsqrt(va_ref[...] + 1e-5)
    t = (t - mu_ref[...]) * inv * g_ref[...] + be_ref[...]
    t = jnp.maximum(t, 0.0)
    _split_store(out_ref, jnp.dot(t, w2_ref[...],
                                  preferred_element_type=jnp.float32) * dis)


def _tc3_body(p_ref, degT_ref, b2_ref, out_ref):
    dis = _dis_of(degT_ref[...])
    out_ref[...] = dis * _cat(p_ref) + b2_ref[...]


_rowspec = pl.BlockSpec((_RB, D), lambda i: (i, 0))
_fullmat = pl.BlockSpec((D, D), lambda i: (0, 0))
_degspec = pl.BlockSpec((_RB, NC), lambda i: (i, 0))
_vecspec = pl.BlockSpec((1, D), lambda i: (0, 0))
_halfspec = pl.BlockSpec((NC, _RB, H), lambda i: (0, i, 0))
_packspec = pl.BlockSpec((NC, _RB, H // 2), lambda i: (0, i, 0))

_tc1 = pl.pallas_call(
    _tc1_body,
    grid=(_GRID,),
    in_specs=[_rowspec, _fullmat, _degspec],
    out_specs=_packspec,
    out_shape=jax.ShapeDtypeStruct((NC, N, H // 2), jnp.int32),
)

_tc2 = pl.pallas_call(
    _tc2_body,
    grid=(_GRID,),
    in_specs=[_halfspec, _degspec,
              _vecspec, _vecspec, _vecspec, _vecspec, _vecspec, _fullmat],
    out_specs=_packspec,
    out_shape=jax.ShapeDtypeStruct((NC, N, H // 2), jnp.int32),
)

_tc3 = pl.pallas_call(
    _tc3_body,
    grid=(_GRID,),
    in_specs=[_halfspec, _degspec, _vecspec],
    out_specs=_rowspec,
    out_shape=jax.ShapeDtypeStruct((N, D), jnp.float32),
)


def kernel(x, edge_index, edge_weight, W1, b1, bn_gamma, bn_beta, bn_mean,
           bn_var, W2, b2):
    loop = jnp.arange(N, dtype=jnp.int32)
    padn = EPAD - E - N
    pad = (jnp.arange(padn, dtype=jnp.int32) * 37) % N
    srcf = jnp.concatenate([edge_index[0], loop, pad]).reshape(NS, ECH, C)
    dstf = jnp.concatenate([edge_index[1], loop, pad]).reshape(NS, ECH, C)
    ewf = jnp.concatenate([
        edge_weight, jnp.ones((N,), jnp.float32),
        jnp.zeros((padn,), jnp.float32)]).reshape(NS, ECH, C)

    dstd = edge_index[1].reshape(NW, DCH, C)
    ewd = edge_weight.reshape(NW, DCH, C)
    zn = jnp.zeros((N,), jnp.float32)
    znd = jnp.zeros((N, H), jnp.float32)

    b1s = b1.reshape(1, D)
    gs = bn_gamma.reshape(1, D)
    bes = bn_beta.reshape(1, D)
    mus = bn_mean.reshape(1, D)
    vas = bn_var.reshape(1, D)
    b2s = b2.reshape(1, D)

    degP = _deg(dstd, ewd, zn)
    degT = degP.T

    h1 = _tc1(x, W1, degT)
    p1 = _edge(h1.reshape(NC * N, H // 2), srcf, dstf, ewf, znd)
    h2 = _tc2(p1, degT, b1s, gs, bes, mus, vas, W2)
    p2 = _edge(h2.reshape(NC * N, H // 2), srcf, dstf, ewf, znd)
    return _tc3(p2, degT, b2s)
